# Initial kernel scaffold; baseline (speedup 1.0000x reference)
import jax
import jax.numpy as jnp
from jax.experimental import pallas as pl


def kernel(x, edge_index, W1, b1, W2, b2):
    raise NotImplementedError("write your pallas kernel here")


# speedup vs baseline: 14.7255x; 14.7255x over previous
import functools

import jax
import jax.numpy as jnp
from jax import lax
from jax.experimental import pallas as pl
from jax.experimental.pallas import tpu as pltpu
from jax.experimental.pallas import tpu_sc as plsc

N = 10000
D = 128
E = 320000
NC = 2
NS = 16
NW = NC * NS
EPW = E // NW
B = 80
NCHUNK = EPW // B

_MESH = plsc.VectorSubcoreMesh(core_axis_name="c", subcore_axis_name="s")


def _deg_body(dst_hbm, zero_hbm, deg_hbm, idx_v, ones_v, acc):
    c = lax.axis_index("c")
    s = lax.axis_index("s")
    w = c * NS + s

    @pl.when(s == 0)
    def _():
        pltpu.sync_copy(zero_hbm, acc)

    for i in range(B // 16):
        ones_v[pl.ds(i * 16, 16)] = jnp.ones((16,), jnp.float32)
    plsc.subcore_barrier()

    base = w * EPW

    def body(j, carry):
        off = base + j * B
        pltpu.sync_copy(dst_hbm.at[pl.ds(off, B)], idx_v.at[0])
        pltpu.sync_copy(ones_v, acc.at[idx_v.at[0]], add=True)
        return carry

    lax.fori_loop(0, NCHUNK, body, 0)
    plsc.subcore_barrier()

    @pl.when(s == 0)
    def _():
        pltpu.sync_copy(acc, deg_hbm.at[c])


_deg = pl.kernel(
    _deg_body,
    mesh=_MESH,
    out_type=jax.ShapeDtypeStruct((NC, N), jnp.float32),
    scratch_types=[
        pltpu.VMEM((1, B), jnp.int32),
        pltpu.VMEM((B,), jnp.float32),
        pltpu.VMEM_SHARED((N,), jnp.float32),
    ],
)


def _agg_body(t_hbm, src_hbm, dst_hbm, zero_hbm, out_hbm,
              sidx, didx, rows, acc, sem):
    c = lax.axis_index("c")
    s = lax.axis_index("s")
    w = c * NS + s

    @pl.when(s == 0)
    def _():
        pltpu.sync_copy(zero_hbm, acc)
    plsc.subcore_barrier()

    base = w * EPW

    def body(j, carry):
        off = base + j * B
        pltpu.sync_copy(src_hbm.at[pl.ds(off, B)], sidx.at[0])
        pltpu.sync_copy(dst_hbm.at[pl.ds(off, B)], didx.at[0])
        pltpu.async_copy(t_hbm.at[sidx.at[0]], rows, sem).wait()
        pltpu.sync_copy(rows, acc.at[didx.at[0]], add=True)
        return carry

    lax.fori_loop(0, NCHUNK, body, 0)
    plsc.subcore_barrier()

    @pl.when(s == 0)
    def _():
        pltpu.sync_copy(acc, out_hbm.at[c])


_agg = pl.kernel(
    _agg_body,
    mesh=_MESH,
    out_type=jax.ShapeDtypeStruct((NC, N, D), jnp.float32),
    scratch_types=[
        pltpu.VMEM((1, B), jnp.int32),
        pltpu.VMEM((1, B), jnp.int32),
        pltpu.VMEM((B, D), jnp.float32),
        pltpu.VMEM_SHARED((N, D), jnp.float32),
        pltpu.SemaphoreType.DMA,
    ],
)


_BM = 2000


def _prep_kernel(degp_ref, x_ref, dinv_ref, t1_ref):
    deg = degp_ref[0] + degp_ref[1] + 1.0
    dinv = lax.rsqrt(deg)
    dinv_ref[...] = dinv
    t1_ref[...] = x_ref[...] * dinv


def _mid_kernel(p_ref, t1_ref, dinv_ref, w1_ref, b1_ref, w2_ref, t2_ref):
    dinv = dinv_ref[...]
    s1 = dinv * (p_ref[0] + p_ref[1] + t1_ref[...])
    h = jnp.dot(s1, w1_ref[...], preferred_element_type=jnp.float32)
    h = jnp.maximum(h + b1_ref[...], 0.0)
    t2 = jnp.dot(h, w2_ref[...], preferred_element_type=jnp.float32)
    t2_ref[...] = t2 * dinv


def _fin_kernel(q_ref, t2_ref, dinv_ref, b2_ref, out_ref):
    out_ref[...] = (dinv_ref[...] * (q_ref[0] + q_ref[1] + t2_ref[...])
                    + b2_ref[...])


def _prep(degp, x):
    return pl.pallas_call(
        _prep_kernel,
        grid=(N // _BM,),
        in_specs=[
            pl.BlockSpec((NC, _BM, 1), lambda i: (0, i, 0)),
            pl.BlockSpec((_BM, D), lambda i: (i, 0)),
        ],
        out_specs=[
            pl.BlockSpec((_BM, 1), lambda i: (i, 0)),
            pl.BlockSpec((_BM, D), lambda i: (i, 0)),
        ],
        out_shape=[
            jax.ShapeDtypeStruct((N, 1), jnp.float32),
            jax.ShapeDtypeStruct((N, D), jnp.float32),
        ],
    )(degp, x)


def _mid(p, t1, dinv, W1, b1, W2):
    return pl.pallas_call(
        _mid_kernel,
        grid=(N // _BM,),
        in_specs=[
            pl.BlockSpec((NC, _BM, D), lambda i: (0, i, 0)),
            pl.BlockSpec((_BM, D), lambda i: (i, 0)),
            pl.BlockSpec((_BM, 1), lambda i: (i, 0)),
            pl.BlockSpec((D, 2 * D), lambda i: (0, 0)),
            pl.BlockSpec((1, 2 * D), lambda i: (0, 0)),
            pl.BlockSpec((2 * D, D), lambda i: (0, 0)),
        ],
        out_specs=pl.BlockSpec((_BM, D), lambda i: (i, 0)),
        out_shape=jax.ShapeDtypeStruct((N, D), jnp.float32),
    )(p, t1, dinv, W1, b1, W2)


def _fin(q, t2, dinv, b2):
    return pl.pallas_call(
        _fin_kernel,
        grid=(N // _BM,),
        in_specs=[
            pl.BlockSpec((NC, _BM, D), lambda i: (0, i, 0)),
            pl.BlockSpec((_BM, D), lambda i: (i, 0)),
            pl.BlockSpec((_BM, 1), lambda i: (i, 0)),
            pl.BlockSpec((1, D), lambda i: (0, 0)),
        ],
        out_specs=pl.BlockSpec((_BM, D), lambda i: (i, 0)),
        out_shape=jax.ShapeDtypeStruct((N, D), jnp.float32),
    )(q, t2, dinv, b2)


def kernel(x, edge_index, W1, b1, W2, b2):
    ei = edge_index.astype(jnp.int32)
    src = ei[0]
    dst = ei[1]
    zeros_n = jnp.zeros((N,), jnp.float32)
    zeros_nd = jnp.zeros((N, D), jnp.float32)

    degp = _deg(dst, zeros_n)
    dinv, t1 = _prep(degp.reshape(NC, N, 1), x)
    p = _agg(t1, src, dst, zeros_nd)
    t2 = _mid(p, t1, dinv, W1, b1.reshape(1, -1), W2)
    q = _agg(t2, src, dst, zeros_nd)
    return _fin(q, t2, dinv, b2.reshape(1, -1))


# speedup vs baseline: 24.4905x; 1.6631x over previous
import jax
import jax.numpy as jnp
from jax import lax
from jax.experimental import pallas as pl
from jax.experimental.pallas import tpu as pltpu
from jax.experimental.pallas import tpu_sc as plsc

N = 10000
D = 128
E = 320000
NC = 2
NS = 16
NW = NC * NS
B = 128
NCH = 80
EPT = NCH * B
E_PAD = NW * EPT
PAD = E_PAD - E
PADR = 16
NA = 10240
NCHD = 80
HIST = 10240

_MESH = plsc.VectorSubcoreMesh(core_axis_name="c", subcore_axis_name="s")


def _deg_body(dst_hbm, zero_hbm, deg_hbm, didx, ones_v, acc, dsem):
    c = lax.axis_index("c")
    s = lax.axis_index("s")
    w = c * NS + s

    pltpu.sync_copy(dst_hbm.at[w], didx)
    for i in range(B // 16):
        ones_v[pl.ds(i * 16, 16)] = jnp.ones((16,), jnp.float32)

    @pl.when(s == 0)
    def _():
        pltpu.sync_copy(zero_hbm, acc)
    plsc.subcore_barrier()

    def fire(j, carry):
        pltpu.async_copy(ones_v, acc.at[didx.at[j]], dsem, add=True)
        return carry
    lax.fori_loop(0, NCHD, fire, 0)

    def drain(j, carry):
        pltpu.make_async_copy(ones_v, acc.at[didx.at[j]], dsem).wait()
        return carry
    lax.fori_loop(0, NCHD, drain, 0)

    plsc.subcore_barrier()

    @pl.when(s == 0)
    def _():
        pltpu.sync_copy(acc, deg_hbm.at[c])


_deg = pl.kernel(
    _deg_body,
    mesh=_MESH,
    out_type=jax.ShapeDtypeStruct((NC, HIST), jnp.float32),
    scratch_types=[
        pltpu.VMEM((NCHD, B), jnp.int32),
        pltpu.VMEM((B,), jnp.float32),
        pltpu.VMEM_SHARED((HIST,), jnp.float32),
        pltpu.SemaphoreType.DMA,
    ],
)


def _agg_body(t_hbm, src_hbm, dst_hbm, zero_hbm, out_hbm,
              sidx, didx, rows, acc, gsem):
    c = lax.axis_index("c")
    s = lax.axis_index("s")
    w = c * NS + s

    pltpu.sync_copy(src_hbm.at[w], sidx)
    pltpu.sync_copy(dst_hbm.at[w], didx)
    pltpu.sync_copy(zero_hbm.at[pl.ds(s * (NA // NS), NA // NS)],
                    acc.at[pl.ds(s * (NA // NS), NA // NS)])
    plsc.subcore_barrier()

    def outer(j, carry):
        pltpu.async_copy(t_hbm.at[sidx.at[j]], rows, gsem).wait()
        pltpu.sync_copy(rows, acc.at[didx.at[j]], add=True)
        return carry

    lax.fori_loop(0, NCH, outer, 0)
    plsc.subcore_barrier()

    @pl.when(s < NS - 1)
    def _():
        pltpu.sync_copy(acc.at[pl.ds(s * 624, 624)],
                        out_hbm.at[c, pl.ds(s * 624, 624)])

    @pl.when(s == NS - 1)
    def _():
        pltpu.sync_copy(acc.at[pl.ds(9360, 640)],
                        out_hbm.at[c, pl.ds(9360, 640)])


_agg = pl.kernel(
    _agg_body,
    mesh=_MESH,
    out_type=jax.ShapeDtypeStruct((NC, N, D), jnp.float32),
    scratch_types=[
        pltpu.VMEM((NCH, B), jnp.int32),
        pltpu.VMEM((NCH, B), jnp.int32),
        pltpu.VMEM((B, D), jnp.float32),
        pltpu.VMEM_SHARED((NA, D), jnp.float32),
        pltpu.SemaphoreType.DMA,
    ],
)


_BM = 2000


def _prep_kernel(degp_ref, x_ref, dinv_ref, t1_ref):
    deg = jnp.sum(degp_ref[...], axis=0) + 1.0
    dinv = lax.rsqrt(deg)
    dinv_ref[...] = dinv
    t1_ref[...] = x_ref[...] * dinv


def _mid_kernel(p0_ref, p1_ref, t1_ref, dinv_ref, w1_ref, b1_ref,
                w2_ref, t2_ref):
    dinv = dinv_ref[...]
    s1 = dinv * (p0_ref[...] + p1_ref[...] + t1_ref[...])
    h = jnp.dot(s1, w1_ref[...], preferred_element_type=jnp.float32)
    h = jnp.maximum(h + b1_ref[...], 0.0)
    t2_ref[...] = jnp.dot(h, w2_ref[...],
                          preferred_element_type=jnp.float32) * dinv


def _fin_kernel(q0_ref, q1_ref, t2_ref, dinv_ref, b2_ref, out_ref):
    out_ref[...] = (dinv_ref[...] * (q0_ref[...] + q1_ref[...] + t2_ref[...])
                    + b2_ref[...])


_BS_D = pl.BlockSpec((_BM, D), lambda i: (i, 0))
_BS_V = pl.BlockSpec((_BM, 1), lambda i: (i, 0))


def _prep(degp, x):
    return pl.pallas_call(
        _prep_kernel,
        grid=(N // _BM,),
        in_specs=[
            pl.BlockSpec((NC, _BM, 1), lambda i: (0, i, 0)),
            _BS_D,
        ],
        out_specs=[_BS_V, _BS_D],
        out_shape=[
            jax.ShapeDtypeStruct((N, 1), jnp.float32),
            jax.ShapeDtypeStruct((N, D), jnp.float32),
        ],
    )(degp, x)


def _mid(p0, p1, t1, dinv, W1, b1, W2):
    return pl.pallas_call(
        _mid_kernel,
        grid=(N // _BM,),
        in_specs=[
            _BS_D, _BS_D, _BS_D, _BS_V,
            pl.BlockSpec((D, 2 * D), lambda i: (0, 0)),
            pl.BlockSpec((1, 2 * D), lambda i: (0, 0)),
            pl.BlockSpec((2 * D, D), lambda i: (0, 0)),
        ],
        out_specs=_BS_D,
        out_shape=jax.ShapeDtypeStruct((N, D), jnp.float32),
    )(p0, p1, t1, dinv, W1, b1, W2)


def _fin(q0, q1, t2, dinv, b2):
    return pl.pallas_call(
        _fin_kernel,
        grid=(N // _BM,),
        in_specs=[
            _BS_D, _BS_D, _BS_D, _BS_V,
            pl.BlockSpec((1, D), lambda i: (0, 0)),
        ],
        out_specs=_BS_D,
        out_shape=jax.ShapeDtypeStruct((N, D), jnp.float32),
    )(q0, q1, t2, dinv, b2)


def kernel(x, edge_index, W1, b1, W2, b2):
    ei = edge_index.astype(jnp.int32)
    pad_src = (jnp.arange(PAD, dtype=jnp.int32) % PADR)
    pad_dst = N + (jnp.arange(PAD, dtype=jnp.int32) % PADR)
    src = jnp.concatenate([ei[0], pad_src])
    dst = jnp.concatenate([ei[1], pad_dst])
    srcA = src.reshape(NW, NCH, B)
    dstA = dst.reshape(NW, NCH, B)
    zeros_deg = jnp.zeros((HIST,), jnp.float32)
    zeros_acc = jnp.zeros((NA, D), jnp.float32)

    degp = _deg(dstA, zeros_deg)
    degp = degp[:, :N].reshape(NC, N, 1)
    dinv, t1 = _prep(degp, x)
    p = _agg(t1, srcA, dstA, zeros_acc)
    t2 = _mid(p[0], p[1], t1, dinv,
              W1, b1.reshape(1, -1), W2)
    q = _agg(t2, srcA, dstA, zeros_acc)
    return _fin(q[0], q[1], t2, dinv, b2.reshape(1, -1))
